# Initial kernel scaffold; baseline (speedup 1.0000x reference)
import jax
import jax.numpy as jnp
from jax.experimental import pallas as pl


def kernel(features, rois):
    raise NotImplementedError("write your pallas kernel here")


# speedup vs baseline: 16.1966x; 16.1966x over previous
import functools

import jax
import jax.numpy as jnp
from jax import lax
from jax.experimental import pallas as pl
from jax.experimental.pallas import tpu as pltpu
from jax.experimental.pallas import tpu_sc as plsc

POOLED_H = 7
POOLED_W = 7
SCALE = 0.0625
B, C, H, W = 2, 768, 14, 14
N = 1000
NW = 32
RPW = 32
G = 4
CC = C // G
NCG = CC // 16
NBIN = POOLED_H * POOLED_W
SPAT = B * H * W
NEG = float(jnp.finfo(jnp.float32).min)


_LO_TAB = [
    [0, 0, 0, 0, 0, 0, 0, 0, 0, 0, 0, 0, 0, 0, 0, 0],
    [0, 0, 0, 0, 0, 0, 0, 1, 1, 1, 1, 1, 1, 1, 2, 2],
    [0, 0, 0, 0, 1, 1, 1, 2, 2, 2, 2, 3, 3, 3, 4, 4],
    [0, 0, 0, 1, 1, 2, 2, 3, 3, 3, 4, 4, 5, 5, 6, 6],
    [0, 0, 1, 1, 2, 2, 3, 4, 4, 5, 5, 6, 6, 7, 8, 8],
    [0, 0, 1, 2, 2, 3, 4, 5, 5, 6, 7, 7, 8, 9, 10, 10],
    [0, 0, 1, 2, 3, 4, 5, 6, 6, 7, 8, 9, 10, 11, 12, 12],
]
_HI_TAB = [
    [0, 1, 1, 1, 1, 1, 1, 1, 2, 2, 2, 2, 2, 2, 2, 3],
    [0, 1, 1, 1, 2, 2, 2, 2, 3, 3, 3, 4, 4, 4, 4, 5],
    [0, 1, 1, 2, 2, 3, 3, 4, 4, 4, 5, 5, 6, 6, 7, 7],
    [0, 1, 2, 2, 3, 3, 4, 4, 5, 6, 6, 7, 7, 8, 8, 9],
    [0, 1, 2, 3, 3, 4, 5, 5, 6, 7, 8, 8, 9, 10, 10, 11],
    [0, 1, 2, 3, 4, 5, 6, 7, 7, 8, 9, 10, 11, 12, 13, 13],
    [0, 1, 2, 3, 4, 5, 6, 7, 8, 9, 10, 11, 12, 13, 14, 14],
]


def _make_params(rois):
    bidx = rois[:, 0].astype(jnp.int32)
    rsw = jnp.round(rois[:, 1] * SCALE).astype(jnp.int32)
    rsh = jnp.round(rois[:, 2] * SCALE).astype(jnp.int32)
    rew = jnp.round(rois[:, 3] * SCALE).astype(jnp.int32)
    reh = jnp.round(rois[:, 4] * SCALE).astype(jnp.int32)
    roi_w = jnp.clip(rew - rsw + 1, 1, 15)
    roi_h = jnp.clip(reh - rsh + 1, 1, 15)
    lo = jnp.asarray(_LO_TAB, jnp.int32)
    hi = jnp.asarray(_HI_TAB, jnp.int32)
    hs = jnp.clip(lo[:, roi_h].T + rsh[:, None], 0, H)
    he = jnp.clip(hi[:, roi_h].T + rsh[:, None], 0, H)
    ws = jnp.clip(lo[:, roi_w].T + rsw[:, None], 0, W)
    we = jnp.clip(hi[:, roi_w].T + rsw[:, None], 0, W)
    params = jnp.concatenate(
        [bidx[:, None], hs, he, ws, we,
         jnp.zeros((N, 3), jnp.int32)], axis=1)
    return jnp.pad(params, ((0, NW * RPW - N), (0, 0)))


def _sc_body(feat_hbm, params_hbm, out_hbm, feat_v, params_v, out_v):
    wid = lax.axis_index("s") * 2 + lax.axis_index("c")
    base = wid * RPW
    cnt = jnp.minimum(RPW, jnp.maximum(N - base, 0))
    pltpu.sync_copy(params_hbm.at[pl.ds(base, RPW)], params_v)
    v49 = lax.iota(jnp.int32, 16) * NBIN

    def chunk_body(g, carry0):
        pltpu.sync_copy(feat_hbm.at[g], feat_v)

        def roi_body(r, carry):
            n = base + r
            row0 = params_v[r, pl.ds(0, 16)]
            row1 = params_v[r, pl.ds(16, 16)]
            posb = row0[0] * (H * W)

            for ph in range(POOLED_H):
                hs = row0[1 + ph]
                he = row0[1 + POOLED_H + ph]
                for pw in range(POOLED_W):
                    wi = 1 + 2 * POOLED_H + pw
                    ws = row0[wi] if wi < 16 else row1[wi - 16]
                    we = row1[wi + POOLED_W - 16]

                    def h_body(h, accs):
                        rowb = posb + h * W

                        def w_body(w, accs2):
                            p = rowb + w
                            return tuple(
                                jnp.maximum(accs2[cg],
                                            feat_v[p, pl.ds(cg * 16, 16)])
                                for cg in range(NCG))

                        return lax.fori_loop(ws, we, w_body, accs)

                    neg = jnp.full((16,), NEG, jnp.float32)
                    accs = lax.fori_loop(hs, he, h_body, (neg,) * NCG)
                    empty = (he <= hs) | (we <= ws)
                    bi = ph * POOLED_W + pw
                    for cg in range(NCG):
                        val = jnp.where(empty, 0.0, accs[cg])
                        plsc.store_scatter(
                            out_v, [v49 + (cg * 16 * NBIN + bi)], val)

            pltpu.sync_copy(out_v, out_hbm.at[n, g])
            return carry

        lax.fori_loop(0, cnt, roi_body, 0)
        return carry0

    lax.fori_loop(0, G, chunk_body, 0)


@jax.jit
def _roi_pool_sc(feats, params):
    mesh = plsc.VectorSubcoreMesh(core_axis_name="c", subcore_axis_name="s")
    f = pl.kernel(
        _sc_body,
        out_type=jax.ShapeDtypeStruct((N, G, CC * NBIN), jnp.float32),
        mesh=mesh,
        compiler_params=pltpu.CompilerParams(needs_layout_passes=False),
        scratch_types=[
            pltpu.VMEM((SPAT, CC), jnp.float32),
            pltpu.VMEM((RPW, 32), jnp.int32),
            pltpu.VMEM((CC * NBIN,), jnp.float32),
        ],
    )
    return f(feats, params)


def kernel(features, rois):
    params = _make_params(rois)
    feats = (features.transpose(0, 2, 3, 1).reshape(SPAT, G, CC)
             .transpose(1, 0, 2))
    out = _roi_pool_sc(feats, params)
    return out.reshape(N, C, POOLED_H, POOLED_W)


# speedup vs baseline: 31.8610x; 1.9671x over previous
import jax
import jax.numpy as jnp
from jax import lax
from jax.experimental import pallas as pl
from jax.experimental.pallas import tpu as pltpu
from jax.experimental.pallas import tpu_sc as plsc

POOLED_H = 7
POOLED_W = 7
SCALE = 0.0625
B, C, H, W = 2, 768, 14, 14
N = 1000
NW = 32
RPW = 32
G = 2
CC = C // G
NCG = CC // 16
NBIN = POOLED_H * POOLED_W
HW = H * W
NEG = float(jnp.finfo(jnp.float32).min)

_LO_TAB = [
    [0, 0, 0, 0, 0, 0, 0, 0, 0, 0, 0, 0, 0, 0, 0, 0],
    [0, 0, 0, 0, 0, 0, 0, 1, 1, 1, 1, 1, 1, 1, 2, 2],
    [0, 0, 0, 0, 1, 1, 1, 2, 2, 2, 2, 3, 3, 3, 4, 4],
    [0, 0, 0, 1, 1, 2, 2, 3, 3, 3, 4, 4, 5, 5, 6, 6],
    [0, 0, 1, 1, 2, 2, 3, 4, 4, 5, 5, 6, 6, 7, 8, 8],
    [0, 0, 1, 2, 2, 3, 4, 5, 5, 6, 7, 7, 8, 9, 10, 10],
    [0, 0, 1, 2, 3, 4, 5, 6, 6, 7, 8, 9, 10, 11, 12, 12],
]
_HI_TAB = [
    [0, 1, 1, 1, 1, 1, 1, 1, 2, 2, 2, 2, 2, 2, 2, 3],
    [0, 1, 1, 1, 2, 2, 2, 2, 3, 3, 3, 4, 4, 4, 4, 5],
    [0, 1, 1, 2, 2, 3, 3, 4, 4, 4, 5, 5, 6, 6, 7, 7],
    [0, 1, 2, 2, 3, 3, 4, 4, 5, 6, 6, 7, 7, 8, 8, 9],
    [0, 1, 2, 3, 3, 4, 5, 5, 6, 7, 8, 8, 9, 10, 10, 11],
    [0, 1, 2, 3, 4, 5, 6, 7, 7, 8, 9, 10, 11, 12, 13, 13],
    [0, 1, 2, 3, 4, 5, 6, 7, 8, 9, 10, 11, 12, 13, 14, 14],
]


def _make_params(rois):
    bidx = rois[:, 0].astype(jnp.int32)
    rsw = jnp.round(rois[:, 1] * SCALE).astype(jnp.int32)
    rsh = jnp.round(rois[:, 2] * SCALE).astype(jnp.int32)
    rew = jnp.round(rois[:, 3] * SCALE).astype(jnp.int32)
    reh = jnp.round(rois[:, 4] * SCALE).astype(jnp.int32)
    roi_w = jnp.clip(rew - rsw + 1, 1, 15)
    roi_h = jnp.clip(reh - rsh + 1, 1, 15)
    lo = jnp.asarray(_LO_TAB, jnp.int32)
    hi = jnp.asarray(_HI_TAB, jnp.int32)
    hs = jnp.clip(lo[:, roi_h].T + rsh[:, None], 0, H)
    he = jnp.clip(hi[:, roi_h].T + rsh[:, None], 0, H)
    ws = jnp.clip(lo[:, roi_w].T + rsw[:, None], 0, W)
    we = jnp.clip(hi[:, roi_w].T + rsw[:, None], 0, W)
    perm = jnp.argsort(bidx, stable=True).astype(jnp.int32)
    n0 = jnp.sum(bidx == 0).astype(jnp.int32)
    hs, he, ws, we = hs[perm], he[perm], ws[perm], we[perm]
    phrows = jnp.concatenate(
        [hs[:, :, None], he[:, :, None],
         jnp.broadcast_to(ws[:, None, :], (N, POOLED_H, POOLED_W)),
         jnp.broadcast_to(we[:, None, :], (N, POOLED_H, POOLED_W))],
        axis=2)
    meta = jnp.concatenate(
        [perm[:, None], jnp.broadcast_to(n0, (N, 1)),
         jnp.zeros((N, 14), jnp.int32)], axis=1)[:, None, :]
    params = jnp.concatenate([phrows, meta], axis=1)
    return jnp.pad(params, ((0, NW * RPW - N), (0, 0), (0, 0)))


def _sc_body(feat_hbm, params_hbm, out_hbm, feat_v, params_v, out_v):
    wid = lax.axis_index("s") * 2 + lax.axis_index("c")
    base = wid * RPW
    cnt = jnp.minimum(RPW, jnp.maximum(N - base, 0))
    pltpu.sync_copy(params_hbm.at[pl.ds(base, RPW)], params_v)
    rowz0 = params_v[0, POOLED_H, pl.ds(0, 16)]
    n0 = rowz0[1]
    bnd = jnp.clip(n0 - base, 0, cnt)
    v49 = lax.iota(jnp.int32, 16) * NBIN

    def chunk_body(g, carry0):
        for seg in range(2):
            lo_r = 0 if seg == 0 else bnd
            hi_r = bnd if seg == 0 else cnt

            @pl.when(hi_r > lo_r)
            def _():
                pltpu.sync_copy(feat_hbm.at[seg, g], feat_v)

                def roi_body(r, carry):
                    def ph_body(ph, carry2):
                        rowp = params_v[r, ph, pl.ds(0, 16)]
                        hs = rowp[0]
                        he = rowp[1]
                        for pw in range(POOLED_W):
                            ws = rowp[2 + pw]
                            we = rowp[2 + POOLED_W + pw]

                            def h_body(h, accs):
                                rowb = h * W

                                def w_body(w, accs2):
                                    p = rowb + w
                                    return tuple(
                                        jnp.maximum(
                                            accs2[cg],
                                            feat_v[p, pl.ds(cg * 16, 16)])
                                        for cg in range(NCG))

                                return lax.fori_loop(ws, we, w_body, accs)

                            neg = jnp.full((16,), NEG, jnp.float32)
                            accs = lax.fori_loop(hs, he, h_body, (neg,) * NCG)
                            empty = (he <= hs) | (we <= ws)
                            bi = ph * POOLED_W + pw
                            for cg in range(NCG):
                                val = jnp.where(empty, 0.0, accs[cg])
                                plsc.store_scatter(
                                    out_v, [v49 + (cg * 16 * NBIN + bi)], val)
                        return carry2

                    lax.fori_loop(0, POOLED_H, ph_body, 0)
                    rowz = params_v[r, POOLED_H, pl.ds(0, 16)]
                    nidx = rowz[0]
                    pltpu.sync_copy(
                        out_v, out_hbm.at[nidx, pl.ds(g * CC * NBIN, CC * NBIN)])
                    return carry

                lax.fori_loop(lo_r, hi_r, roi_body, 0)

        return carry0

    lax.fori_loop(0, G, chunk_body, 0)


@jax.jit
def _roi_pool_sc(feats, params):
    mesh = plsc.VectorSubcoreMesh(core_axis_name="c", subcore_axis_name="s")
    f = pl.kernel(
        _sc_body,
        out_type=jax.ShapeDtypeStruct((N, C * NBIN), jnp.float32),
        mesh=mesh,
        compiler_params=pltpu.CompilerParams(needs_layout_passes=False),
        scratch_types=[
            pltpu.VMEM((HW, CC), jnp.float32),
            pltpu.VMEM((RPW, POOLED_H + 1, 16), jnp.int32),
            pltpu.VMEM((CC * NBIN,), jnp.float32),
        ],
    )
    return f(feats, params)


def kernel(features, rois):
    params = _make_params(rois)
    feats = (features.transpose(0, 2, 3, 1).reshape(B, HW, G, CC)
             .transpose(0, 2, 1, 3))
    out = _roi_pool_sc(feats, params)
    return out.reshape(N, C, POOLED_H, POOLED_W)


# speedup vs baseline: 43.5728x; 1.3676x over previous
import jax
import jax.numpy as jnp
from jax import lax
from jax.experimental import pallas as pl
from jax.experimental.pallas import tpu as pltpu
from jax.experimental.pallas import tpu_sc as plsc

POOLED_H = 7
POOLED_W = 7
SCALE = 0.0625
B, C, H, W = 2, 768, 14, 14
N = 1000
NW = 32
RPW = 32
G = 2
CC = C // G
NCG = CC // 16
NBIN = POOLED_H * POOLED_W
HW = H * W
NEG = float(jnp.finfo(jnp.float32).min)

_LO_TAB = [
    [0, 0, 0, 0, 0, 0, 0, 0, 0, 0, 0, 0, 0, 0, 0, 0],
    [0, 0, 0, 0, 0, 0, 0, 1, 1, 1, 1, 1, 1, 1, 2, 2],
    [0, 0, 0, 0, 1, 1, 1, 2, 2, 2, 2, 3, 3, 3, 4, 4],
    [0, 0, 0, 1, 1, 2, 2, 3, 3, 3, 4, 4, 5, 5, 6, 6],
    [0, 0, 1, 1, 2, 2, 3, 4, 4, 5, 5, 6, 6, 7, 8, 8],
    [0, 0, 1, 2, 2, 3, 4, 5, 5, 6, 7, 7, 8, 9, 10, 10],
    [0, 0, 1, 2, 3, 4, 5, 6, 6, 7, 8, 9, 10, 11, 12, 12],
]
_HI_TAB = [
    [0, 1, 1, 1, 1, 1, 1, 1, 2, 2, 2, 2, 2, 2, 2, 3],
    [0, 1, 1, 1, 2, 2, 2, 2, 3, 3, 3, 4, 4, 4, 4, 5],
    [0, 1, 1, 2, 2, 3, 3, 4, 4, 4, 5, 5, 6, 6, 7, 7],
    [0, 1, 2, 2, 3, 3, 4, 4, 5, 6, 6, 7, 7, 8, 8, 9],
    [0, 1, 2, 3, 3, 4, 5, 5, 6, 7, 8, 8, 9, 10, 10, 11],
    [0, 1, 2, 3, 4, 5, 6, 7, 7, 8, 9, 10, 11, 12, 13, 13],
    [0, 1, 2, 3, 4, 5, 6, 7, 8, 9, 10, 11, 12, 13, 14, 14],
]


def _make_params(rois):
    bidx = rois[:, 0].astype(jnp.int32)
    rsw = jnp.round(rois[:, 1] * SCALE).astype(jnp.int32)
    rsh = jnp.round(rois[:, 2] * SCALE).astype(jnp.int32)
    rew = jnp.round(rois[:, 3] * SCALE).astype(jnp.int32)
    reh = jnp.round(rois[:, 4] * SCALE).astype(jnp.int32)
    roi_w = jnp.clip(rew - rsw + 1, 1, 15)
    roi_h = jnp.clip(reh - rsh + 1, 1, 15)
    lo = jnp.asarray(_LO_TAB, jnp.int32)
    hi = jnp.asarray(_HI_TAB, jnp.int32)
    hs = jnp.clip(lo[:, roi_h].T + rsh[:, None], 0, H)
    he = jnp.clip(hi[:, roi_h].T + rsh[:, None], 0, H)
    ws = jnp.clip(lo[:, roi_w].T + rsw[:, None], 0, W)
    we = jnp.clip(hi[:, roi_w].T + rsw[:, None], 0, W)
    perm = jnp.argsort(bidx, stable=True).astype(jnp.int32)
    n0 = jnp.sum(bidx == 0).astype(jnp.int32)
    hs, he, ws, we = hs[perm], he[perm], ws[perm], we[perm]
    phrows = jnp.concatenate(
        [hs[:, :, None], he[:, :, None],
         jnp.broadcast_to(ws[:, None, :], (N, POOLED_H, POOLED_W)),
         jnp.broadcast_to(we[:, None, :], (N, POOLED_H, POOLED_W))],
        axis=2)
    meta = jnp.concatenate(
        [perm[:, None], jnp.broadcast_to(n0, (N, 1)),
         jnp.zeros((N, 14), jnp.int32)], axis=1)[:, None, :]
    params = jnp.concatenate([phrows, meta], axis=1)
    return jnp.pad(params, ((0, NW * RPW - N), (0, 0), (0, 0)))


def _sc_body(feat_hbm, params_hbm, out_hbm, feat_v, params_v, stage_v, sem):
    wid = lax.axis_index("s") * 2 + lax.axis_index("c")
    base = wid * RPW
    cnt = jnp.minimum(RPW, jnp.maximum(N - base, 0))
    pltpu.sync_copy(params_hbm.at[pl.ds(base, RPW)], params_v)
    rowz0 = params_v[0, POOLED_H, pl.ds(0, 16)]
    n0 = rowz0[1]
    bnd = jnp.clip(n0 - base, 0, cnt)

    def wait_one():
        pltpu.make_async_copy(
            stage_v.at[0], out_hbm.at[0, 0, 0], sem).wait()

    def chunk_body(g, carry0):
        for seg in range(2):
            lo_r = 0 if seg == 0 else bnd
            hi_r = bnd if seg == 0 else cnt

            @pl.when(hi_r > lo_r)
            def _():
                pltpu.sync_copy(feat_hbm.at[seg, g], feat_v)

                def roi_body(r, t):
                    rowz = params_v[r, POOLED_H, pl.ds(0, 16)]
                    nidx = rowz[0]

                    def ph_body(ph, t2):
                        @pl.when(t2 >= 2)
                        def _w():
                            wait_one()

                        par = t2 % 2
                        rowp = params_v[r, ph, pl.ds(0, 16)]
                        hs = rowp[0]
                        he = rowp[1]
                        for pw in range(POOLED_W):
                            ws = rowp[2 + pw]
                            we = rowp[2 + POOLED_W + pw]

                            def h_body(h, accs):
                                rowb = h * W

                                def w_body(w, accs2):
                                    p = rowb + w
                                    return tuple(
                                        jnp.maximum(
                                            accs2[cg],
                                            feat_v[p, pl.ds(cg * 16, 16)])
                                        for cg in range(NCG))

                                return lax.fori_loop(ws, we, w_body, accs)

                            neg = jnp.full((16,), NEG, jnp.float32)
                            accs = lax.fori_loop(hs, he, h_body, (neg,) * NCG)
                            empty = (he <= hs) | (we <= ws)
                            for cg in range(NCG):
                                val = jnp.where(empty, 0.0, accs[cg])
                                stage_v[par, pw, pl.ds(cg * 16, 16)] = val

                        pltpu.async_copy(
                            stage_v.at[par], out_hbm.at[nidx, ph, g], sem)
                        return t2 + 1

                    return lax.fori_loop(0, POOLED_H, ph_body, t)

                t_end = lax.fori_loop(lo_r, hi_r, roi_body, 0)

                @pl.when(t_end >= 1)
                def _d1():
                    wait_one()

                @pl.when(t_end >= 2)
                def _d2():
                    wait_one()

        return carry0

    lax.fori_loop(0, G, chunk_body, 0)


@jax.jit
def _roi_pool_sc(feats, params):
    mesh = plsc.VectorSubcoreMesh(core_axis_name="c", subcore_axis_name="s")
    f = pl.kernel(
        _sc_body,
        out_type=jax.ShapeDtypeStruct((N, POOLED_H, G, POOLED_W, CC),
                                      jnp.float32),
        mesh=mesh,
        compiler_params=pltpu.CompilerParams(needs_layout_passes=False),
        scratch_types=[
            pltpu.VMEM((HW, CC), jnp.float32),
            pltpu.VMEM((RPW, POOLED_H + 1, 16), jnp.int32),
            pltpu.VMEM((2, POOLED_W, CC), jnp.float32),
            pltpu.SemaphoreType.DMA,
        ],
    )
    return f(feats, params)


def kernel(features, rois):
    params = _make_params(rois)
    feats = (features.transpose(0, 2, 3, 1).reshape(B, HW, G, CC)
             .transpose(0, 2, 1, 3))
    out = _roi_pool_sc(feats, params)
    return out.transpose(0, 2, 4, 1, 3).reshape(N, C, POOLED_H, POOLED_W)
